# baseline (speedup 1.0000x reference)
import functools

import jax
import jax.numpy as jnp
from jax import lax
from jax.experimental import pallas as pl
from jax.experimental.pallas import tpu as pltpu
from jax.experimental.pallas import tpu_sc as plsc

_N = 10000
_D = 128
_E = 320000
_G = 64
_OUT = 10

_NW = 32
_NPAD = 10112
_ZROWS = _NPAD // 16
_CHUNK = 128
_CH0 = 56
_CH1 = 104
_CHMAX = max(_CH0, _CH1)
_TOTCH = 16 * (_CH0 + _CH1)
_TOTCH_ALLOC = _TOTCH + _CHMAX
_EPAD = _TOTCH * _CHUNK

_mesh = plsc.VectorSubcoreMesh(core_axis_name="c", subcore_axis_name="s")


@functools.partial(
    pl.kernel,
    out_type=jax.ShapeDtypeStruct((2, _NPAD, _D), jnp.float32),
    mesh=_mesh,
    scratch_types=[
        pltpu.VMEM((_CHMAX, _CHUNK), jnp.int32),
        pltpu.VMEM((_CHMAX, _CHUNK), jnp.int32),
        pltpu.VMEM((_CHUNK, _D), jnp.float32),
        pltpu.VMEM_SHARED((_NPAD, _D), jnp.float32),
        pltpu.SemaphoreType.DMA,
    ],
)
def _sc_agg(x_hbm, src_hbm, dst_hbm, z_hbm, out_hbm, srcv, dstv, rows, agg_sh,
            sem):
    cid = lax.axis_index("c")
    sid = lax.axis_index("s")
    base = jnp.where(cid == 0, sid * _CH0, 16 * _CH0 + sid * _CH1)
    nch = jnp.where(cid == 0, _CH0, _CH1)
    pltpu.sync_copy(z_hbm, agg_sh.at[pl.ds(sid * _ZROWS, _ZROWS)])
    pltpu.sync_copy(src_hbm.at[pl.ds(base, _CHMAX)], srcv)
    pltpu.sync_copy(dst_hbm.at[pl.ds(base, _CHMAX)], dstv)
    plsc.subcore_barrier()

    @pl.loop(0, nch)
    def _edge_chunks(j):
        pltpu.async_copy(x_hbm.at[srcv.at[j]], rows, sem).wait()
        pltpu.sync_copy(rows, agg_sh.at[dstv.at[j]], add=True)

    plsc.subcore_barrier()
    pltpu.sync_copy(agg_sh.at[pl.ds(sid * _ZROWS, _ZROWS)],
                    out_hbm.at[cid, pl.ds(sid * _ZROWS, _ZROWS)])


def _layer_body(h_ref, agg_ref, w1_ref, b1_ref, w2_ref, b2_ref, g_ref, be_ref, o_ref):
    u = h_ref[...] + agg_ref[0] + agg_ref[1]
    t = jnp.dot(u, w1_ref[...], preferred_element_type=jnp.float32) + b1_ref[...]
    t = jnp.where(t > 0, t, 0.01 * t)
    v = jnp.dot(t, w2_ref[...], preferred_element_type=jnp.float32) + b2_ref[...]
    v = jnp.where(v > 0, v, 0.01 * v)
    m = jnp.mean(v, axis=0, keepdims=True)
    c = v - m
    var = jnp.mean(c * c, axis=0, keepdims=True)
    o_ref[...] = c * lax.rsqrt(var + 1e-5) * g_ref[...] + be_ref[...]


_layer_call = pl.pallas_call(
    _layer_body,
    out_shape=jax.ShapeDtypeStruct((_N, _D), jnp.float32),
)


def _head_body(h_ref, batch_ref, wh1_ref, bh1_ref, wh2_ref, bh2_ref, o_ref):
    onehot = (batch_ref[...] == lax.broadcasted_iota(jnp.int32, (1, _G), 1)
              ).astype(jnp.float32)
    pooled = lax.dot_general(onehot, h_ref[...], (((0,), (0,)), ((), ())),
                             preferred_element_type=jnp.float32)
    counts = jnp.sum(onehot, axis=0)[:, None]
    pooled = pooled / jnp.maximum(counts, 1.0)
    t = jnp.dot(pooled, wh1_ref[...], preferred_element_type=jnp.float32) + bh1_ref[...]
    t = jnp.where(t > 0, t, 0.01 * t)
    o_ref[...] = jnp.dot(t, wh2_ref[...], preferred_element_type=jnp.float32) + bh2_ref[...]


_head_call = pl.pallas_call(
    _head_body,
    out_shape=jax.ShapeDtypeStruct((_G, _OUT), jnp.float32),
)


def kernel(x, edge_index, batch,
           W1_0, b1_0, W2_0, b2_0, g_0, be_0,
           W1_1, b1_1, W2_1, b2_1, g_1, be_1,
           W1_2, b1_2, W2_2, b2_2, g_2, be_2,
           Wh1, bh1, Wh2, bh2):
    src = edge_index[0]
    dst = edge_index[1]
    allocpad = _TOTCH_ALLOC * _CHUNK - _E
    srcp = jnp.concatenate([src, jnp.zeros((allocpad,), jnp.int32)]
                           ).reshape(_TOTCH_ALLOC, _CHUNK)
    dstp = jnp.concatenate([dst, jnp.full((allocpad,), _NPAD - 1, jnp.int32)]
                           ).reshape(_TOTCH_ALLOC, _CHUNK)
    zeros = jnp.zeros((_ZROWS, _D), jnp.float32)

    layers = [
        (W1_0, b1_0, W2_0, b2_0, g_0, be_0),
        (W1_1, b1_1, W2_1, b2_1, g_1, be_1),
        (W1_2, b1_2, W2_2, b2_2, g_2, be_2),
    ]
    h = x
    for (W1, b1, W2, b2, g, be) in layers:
        agg = _sc_agg(h, srcp, dstp, zeros)[:, :_N, :]
        h = _layer_call(h, agg, W1, b1.reshape(1, _D), W2, b2.reshape(1, _D),
                        g.reshape(1, _D), be.reshape(1, _D))
    return _head_call(h, batch.reshape(_N, 1), Wh1, bh1.reshape(1, _D),
                      Wh2, bh2.reshape(1, _OUT))


# speedup vs baseline: 1.1334x; 1.1334x over previous
import functools

import jax
import jax.numpy as jnp
from jax import lax
from jax.experimental import pallas as pl
from jax.experimental.pallas import tpu as pltpu
from jax.experimental.pallas import tpu_sc as plsc

_N = 10000
_D = 128
_E = 320000
_G = 64
_OUT = 10

_NW = 32
_NPAD = 10112
_ZROWS = _NPAD // 16
_CHUNK = 128
_BLK = 10
_NBLK = 8
_CH = _NBLK * _BLK
_EPT = _E // _NW
_EPAD = _NW * _CH * _CHUNK

_mesh = plsc.VectorSubcoreMesh(core_axis_name="c", subcore_axis_name="s")


@functools.partial(
    pl.kernel,
    out_type=jax.ShapeDtypeStruct((2, _NPAD, _D), jnp.float32),
    mesh=_mesh,
    scratch_types=[
        pltpu.VMEM((2, _BLK, _CHUNK), jnp.int32),
        pltpu.VMEM((2, _BLK, _CHUNK), jnp.int32),
        pltpu.VMEM((2, _CHUNK, _D), jnp.float32),
        pltpu.VMEM((1, _CHUNK), jnp.int32),
        pltpu.VMEM_SHARED((_NPAD, _D), jnp.float32),
        (pltpu.SemaphoreType.DMA, pltpu.SemaphoreType.DMA),
        (pltpu.SemaphoreType.DMA, pltpu.SemaphoreType.DMA),
        (pltpu.SemaphoreType.DMA, pltpu.SemaphoreType.DMA),
    ],
)
def _sc_agg(x_hbm, src_hbm, dst_hbm, z_hbm, sink_hbm, out_hbm,
            srcv, dstv, rows, sinkv, agg_sh, gsem, ssem, isem):
    cid = lax.axis_index("c")
    sid = lax.axis_index("s")
    wid = sid * 2 + cid

    def fire_idx(blk, slot):
        pltpu.async_copy(src_hbm.at[wid, blk], srcv.at[slot], isem[slot])
        pltpu.async_copy(dst_hbm.at[wid, blk], dstv.at[slot], isem[slot])

    def wait_idx(blk, slot):
        pltpu.make_async_copy(src_hbm.at[wid, blk], srcv.at[slot],
                              isem[slot]).wait()
        pltpu.make_async_copy(dst_hbm.at[wid, blk], dstv.at[slot],
                              isem[slot]).wait()

    def fire_gather(s, c, buf):
        pltpu.async_copy(x_hbm.at[srcv.at[s, c]], rows.at[buf], gsem[buf])

    def wait_gather(s, c, buf):
        pltpu.make_async_copy(x_hbm.at[srcv.at[s, c]], rows.at[buf],
                              gsem[buf]).wait()

    def fire_scatter(s, c, buf):
        pltpu.async_copy(rows.at[buf], agg_sh.at[dstv.at[s, c]], ssem[buf],
                         add=True)

    def wait_scatter(buf):
        pltpu.make_async_copy(rows.at[buf], agg_sh.at[sinkv.at[0]],
                              ssem[buf]).wait()

    fire_idx(0, 0)
    pltpu.sync_copy(z_hbm, agg_sh.at[pl.ds(sid * _ZROWS, _ZROWS)])
    pltpu.sync_copy(sink_hbm, sinkv)
    wait_idx(0, 0)
    fire_gather(0, 0, 0)
    pltpu.async_copy(rows.at[1], agg_sh.at[sinkv.at[0]], ssem[1], add=True)
    plsc.subcore_barrier()

    @pl.loop(0, _NBLK, step=2)
    def _blocks(b0):
        for s in range(2):
            bb = b0 + s

            @pl.loop(0, _BLK // 2)
            def _pairs(t):
                for half in range(2):
                    c = 2 * t + half
                    wait_gather(s, c, half)
                    fire_scatter(s, c, half)
                    wait_scatter(1 - half)
                    if half == 0:
                        fire_gather(s, c + 1, 1)
                    else:
                        @pl.when(c + 1 < _BLK)
                        def _():
                            fire_gather(s, c + 1, 0)

                @pl.when(jnp.logical_and(t == 0, bb + 1 < _NBLK))
                def _():
                    fire_idx(bb + 1, 1 - s)

            @pl.when(bb + 1 < _NBLK)
            def _():
                wait_idx(bb + 1, 1 - s)
                fire_gather(1 - s, 0, 0)

    wait_scatter(1)
    plsc.subcore_barrier()
    pltpu.sync_copy(agg_sh.at[pl.ds(sid * _ZROWS, _ZROWS)],
                    out_hbm.at[cid, pl.ds(sid * _ZROWS, _ZROWS)])


def _layer_body(h_ref, agg_ref, w1_ref, b1_ref, w2_ref, b2_ref, g_ref, be_ref, o_ref):
    u = h_ref[...] + agg_ref[0] + agg_ref[1]
    t = jnp.dot(u, w1_ref[...], preferred_element_type=jnp.float32) + b1_ref[...]
    t = jnp.where(t > 0, t, 0.01 * t)
    v = jnp.dot(t, w2_ref[...], preferred_element_type=jnp.float32) + b2_ref[...]
    v = jnp.where(v > 0, v, 0.01 * v)
    m = jnp.mean(v, axis=0, keepdims=True)
    c = v - m
    var = jnp.mean(c * c, axis=0, keepdims=True)
    o_ref[...] = c * lax.rsqrt(var + 1e-5) * g_ref[...] + be_ref[...]


_layer_call = pl.pallas_call(
    _layer_body,
    out_shape=jax.ShapeDtypeStruct((_N, _D), jnp.float32),
)


def _head_body(h_ref, batch_ref, wh1_ref, bh1_ref, wh2_ref, bh2_ref, o_ref):
    onehot = (batch_ref[...] == lax.broadcasted_iota(jnp.int32, (1, _G), 1)
              ).astype(jnp.float32)
    pooled = lax.dot_general(onehot, h_ref[...], (((0,), (0,)), ((), ())),
                             preferred_element_type=jnp.float32)
    counts = jnp.sum(onehot, axis=0)[:, None]
    pooled = pooled / jnp.maximum(counts, 1.0)
    t = jnp.dot(pooled, wh1_ref[...], preferred_element_type=jnp.float32) + bh1_ref[...]
    t = jnp.where(t > 0, t, 0.01 * t)
    o_ref[...] = jnp.dot(t, wh2_ref[...], preferred_element_type=jnp.float32) + bh2_ref[...]


_head_call = pl.pallas_call(
    _head_body,
    out_shape=jax.ShapeDtypeStruct((_G, _OUT), jnp.float32),
)


def kernel(x, edge_index, batch,
           W1_0, b1_0, W2_0, b2_0, g_0, be_0,
           W1_1, b1_1, W2_1, b2_1, g_1, be_1,
           W1_2, b1_2, W2_2, b2_2, g_2, be_2,
           Wh1, bh1, Wh2, bh2):
    src = edge_index[0]
    dst = edge_index[1]
    allocpad = _EPAD - _E
    srcp = jnp.concatenate([src, jnp.zeros((allocpad,), jnp.int32)]
                           ).reshape(_NW, _NBLK, _BLK, _CHUNK)
    dstp = jnp.concatenate([dst, jnp.full((allocpad,), _NPAD - 1, jnp.int32)]
                           ).reshape(_NW, _NBLK, _BLK, _CHUNK)
    zeros = jnp.zeros((_ZROWS, _D), jnp.float32)
    sink = jnp.full((1, _CHUNK), _NPAD - 1, jnp.int32)

    layers = [
        (W1_0, b1_0, W2_0, b2_0, g_0, be_0),
        (W1_1, b1_1, W2_1, b2_1, g_1, be_1),
        (W1_2, b1_2, W2_2, b2_2, g_2, be_2),
    ]
    h = x
    for (W1, b1, W2, b2, g, be) in layers:
        agg = _sc_agg(h, srcp, dstp, zeros, sink)[:, :_N, :]
        h = _layer_call(h, agg, W1, b1.reshape(1, _D), W2, b2.reshape(1, _D),
                        g.reshape(1, _D), be.reshape(1, _D))
    return _head_call(h, batch.reshape(_N, 1), Wh1, bh1.reshape(1, _D),
                      Wh2, bh2.reshape(1, _OUT))


# speedup vs baseline: 1.3092x; 1.1552x over previous
import functools

import jax
import jax.numpy as jnp
from jax import lax
from jax.experimental import pallas as pl
from jax.experimental.pallas import tpu as pltpu
from jax.experimental.pallas import tpu_sc as plsc

_N = 10000
_D = 128
_E = 320000
_G = 64
_OUT = 10

_NW = 32
_NPAD = 10112
_ZROWS = _NPAD // 16
_CHUNK = 128
_BLK = 10
_NBLK = 8
_CH = _NBLK * _BLK
_EPT = _E // _NW
_EPAD = _NW * _CH * _CHUNK

_mesh = plsc.VectorSubcoreMesh(core_axis_name="c", subcore_axis_name="s")


@functools.partial(
    pl.kernel,
    out_type=jax.ShapeDtypeStruct((2, _NPAD, _D), jnp.float32),
    mesh=_mesh,
    scratch_types=[
        pltpu.VMEM((2, _BLK, _CHUNK), jnp.int32),
        pltpu.VMEM((2, _BLK, _CHUNK), jnp.int32),
        pltpu.VMEM((2, _CHUNK, _D), jnp.float32),
        pltpu.VMEM((1, _CHUNK), jnp.int32),
        pltpu.VMEM_SHARED((_NPAD, _D), jnp.float32),
        (pltpu.SemaphoreType.DMA, pltpu.SemaphoreType.DMA),
        (pltpu.SemaphoreType.DMA, pltpu.SemaphoreType.DMA),
        (pltpu.SemaphoreType.DMA, pltpu.SemaphoreType.DMA),
    ],
)
def _sc_agg(x_hbm, src_hbm, dst_hbm, z_hbm, sink_hbm, out_hbm,
            srcv, dstv, rows, sinkv, agg_sh, gsem, ssem, isem):
    cid = lax.axis_index("c")
    sid = lax.axis_index("s")
    wid = sid * 2 + cid

    def fire_idx(blk, slot):
        pltpu.async_copy(src_hbm.at[wid, blk], srcv.at[slot], isem[slot])
        pltpu.async_copy(dst_hbm.at[wid, blk], dstv.at[slot], isem[slot])

    def wait_idx(blk, slot):
        pltpu.make_async_copy(src_hbm.at[wid, blk], srcv.at[slot],
                              isem[slot]).wait()
        pltpu.make_async_copy(dst_hbm.at[wid, blk], dstv.at[slot],
                              isem[slot]).wait()

    def fire_gather(s, c, buf):
        pltpu.async_copy(x_hbm.at[srcv.at[s, c]], rows.at[buf], gsem[buf])

    def wait_gather(s, c, buf):
        pltpu.make_async_copy(x_hbm.at[srcv.at[s, c]], rows.at[buf],
                              gsem[buf]).wait()

    def fire_scatter(s, c, buf):
        pltpu.async_copy(rows.at[buf], agg_sh.at[dstv.at[s, c]], ssem[buf],
                         add=True)

    def wait_scatter(buf):
        pltpu.make_async_copy(rows.at[buf], agg_sh.at[sinkv.at[0]],
                              ssem[buf]).wait()

    fire_idx(0, 0)
    pltpu.sync_copy(z_hbm, agg_sh.at[pl.ds(sid * _ZROWS, _ZROWS)])
    pltpu.sync_copy(sink_hbm, sinkv)
    wait_idx(0, 0)
    fire_gather(0, 0, 0)
    pltpu.async_copy(rows.at[1], agg_sh.at[sinkv.at[0]], ssem[1], add=True)
    plsc.subcore_barrier()

    @pl.loop(0, _NBLK, step=2)
    def _blocks(b0):
        for s in range(2):
            bb = b0 + s

            @pl.loop(0, _BLK // 2)
            def _pairs(t):
                for half in range(2):
                    c = 2 * t + half
                    wait_gather(s, c, half)
                    fire_scatter(s, c, half)
                    wait_scatter(1 - half)
                    if half == 0:
                        fire_gather(s, c + 1, 1)
                    else:
                        @pl.when(c + 1 < _BLK)
                        def _():
                            fire_gather(s, c + 1, 0)

                @pl.when(jnp.logical_and(t == 0, bb + 1 < _NBLK))
                def _():
                    fire_idx(bb + 1, 1 - s)

            @pl.when(bb + 1 < _NBLK)
            def _():
                wait_idx(bb + 1, 1 - s)
                fire_gather(1 - s, 0, 0)

    wait_scatter(1)
    plsc.subcore_barrier()
    pltpu.sync_copy(agg_sh.at[pl.ds(sid * _ZROWS, _ZROWS)],
                    out_hbm.at[cid, pl.ds(sid * _ZROWS, _ZROWS)])


def _layer_body(h_ref, agg_ref, w1_ref, b1_ref, w2_ref, b2_ref, g_ref, be_ref, o_ref):
    u = h_ref[...] + agg_ref[0] + agg_ref[1]
    t = jnp.dot(u, w1_ref[...], preferred_element_type=jnp.float32) + b1_ref[...]
    t = jnp.where(t > 0, t, 0.01 * t)
    v = jnp.dot(t, w2_ref[...], preferred_element_type=jnp.float32) + b2_ref[...]
    v = jnp.where(v > 0, v, 0.01 * v)
    m = jnp.mean(v, axis=0, keepdims=True)
    c = v - m
    var = jnp.mean(c * c, axis=0, keepdims=True)
    o_ref[...] = c * lax.rsqrt(var + 1e-5) * g_ref[...] + be_ref[...]


_layer_call = pl.pallas_call(
    _layer_body,
    out_shape=jax.ShapeDtypeStruct((_N, _D), jnp.float32),
)


def _head_body(h_ref, batch_ref, wh1_ref, bh1_ref, wh2_ref, bh2_ref, o_ref):
    onehot = (batch_ref[...] == lax.broadcasted_iota(jnp.int32, (1, _G), 1)
              ).astype(jnp.float32)
    pooled = lax.dot_general(onehot, h_ref[...], (((0,), (0,)), ((), ())),
                             preferred_element_type=jnp.float32)
    counts = jnp.sum(onehot, axis=0)[:, None]
    pooled = pooled / jnp.maximum(counts, 1.0)
    t = jnp.dot(pooled, wh1_ref[...], preferred_element_type=jnp.float32) + bh1_ref[...]
    t = jnp.where(t > 0, t, 0.01 * t)
    o_ref[...] = jnp.dot(t, wh2_ref[...], preferred_element_type=jnp.float32) + bh2_ref[...]


_head_call = pl.pallas_call(
    _head_body,
    out_shape=jax.ShapeDtypeStruct((_G, _OUT), jnp.float32),
)


def kernel(x, edge_index, batch,
           W1_0, b1_0, W2_0, b2_0, g_0, be_0,
           W1_1, b1_1, W2_1, b2_1, g_1, be_1,
           W1_2, b1_2, W2_2, b2_2, g_2, be_2,
           Wh1, bh1, Wh2, bh2):
    src = edge_index[0]
    dst = edge_index[1]
    pad = _CH * _CHUNK - _EPT
    sink_rows = _N + (jnp.arange(_NW * pad, dtype=jnp.int32) % (_NPAD - _N))
    srcp = jnp.concatenate(
        [src.reshape(_NW, _EPT), jnp.zeros((_NW, pad), jnp.int32)], axis=1,
    ).reshape(_NW, _NBLK, _BLK, _CHUNK)
    dstp = jnp.concatenate(
        [dst.reshape(_NW, _EPT), sink_rows.reshape(_NW, pad)], axis=1,
    ).reshape(_NW, _NBLK, _BLK, _CHUNK)
    zeros = jnp.zeros((_ZROWS, _D), jnp.float32)
    sink = (_N + (jnp.arange(_CHUNK, dtype=jnp.int32) % (_NPAD - _N))
            ).reshape(1, _CHUNK)

    layers = [
        (W1_0, b1_0, W2_0, b2_0, g_0, be_0),
        (W1_1, b1_1, W2_1, b2_1, g_1, be_1),
        (W1_2, b1_2, W2_2, b2_2, g_2, be_2),
    ]
    h = x
    for (W1, b1, W2, b2, g, be) in layers:
        agg = _sc_agg(h, srcp, dstp, zeros, sink)[:, :_N, :]
        h = _layer_call(h, agg, W1, b1.reshape(1, _D), W2, b2.reshape(1, _D),
                        g.reshape(1, _D), be.reshape(1, _D))
    return _head_call(h, batch.reshape(_N, 1), Wh1, bh1.reshape(1, _D),
                      Wh2, bh2.reshape(1, _OUT))


# speedup vs baseline: 1.3356x; 1.0201x over previous
import functools

import jax
import jax.numpy as jnp
from jax import lax
from jax.experimental import pallas as pl
from jax.experimental.pallas import tpu as pltpu
from jax.experimental.pallas import tpu_sc as plsc

_N = 10000
_D = 128
_E = 320000
_G = 64
_OUT = 10

_NW = 32
_NPAD = 10112
_ZROWS = _NPAD // 16
_CHUNK = 64
_BLK = 8
_NBLK = 20
_CH = _NBLK * _BLK
_EPT = _E // _NW
_EPAD = _NW * _CH * _CHUNK

_mesh = plsc.VectorSubcoreMesh(core_axis_name="c", subcore_axis_name="s")


@functools.partial(
    pl.kernel,
    out_type=jax.ShapeDtypeStruct((2, _NPAD, _D), jnp.float32),
    mesh=_mesh,
    scratch_types=[
        pltpu.VMEM((2, _BLK, _CHUNK), jnp.int32),
        pltpu.VMEM((2, _BLK, _CHUNK), jnp.int32),
        pltpu.VMEM((4, _CHUNK, _D), jnp.float32),
        pltpu.VMEM((1, _CHUNK), jnp.int32),
        pltpu.VMEM_SHARED((_NPAD, _D), jnp.float32),
        (pltpu.SemaphoreType.DMA, pltpu.SemaphoreType.DMA,
         pltpu.SemaphoreType.DMA, pltpu.SemaphoreType.DMA),
        (pltpu.SemaphoreType.DMA, pltpu.SemaphoreType.DMA,
         pltpu.SemaphoreType.DMA, pltpu.SemaphoreType.DMA),
        (pltpu.SemaphoreType.DMA, pltpu.SemaphoreType.DMA),
    ],
)
def _sc_agg(x_hbm, src_hbm, dst_hbm, z_hbm, sink_hbm, out_hbm,
            srcv, dstv, rows, sinkv, agg_sh, gsem, ssem, isem):
    cid = lax.axis_index("c")
    sid = lax.axis_index("s")
    wid = sid * 2 + cid

    def fire_idx(blk, slot):
        pltpu.async_copy(src_hbm.at[wid, blk], srcv.at[slot], isem[slot])
        pltpu.async_copy(dst_hbm.at[wid, blk], dstv.at[slot], isem[slot])

    def wait_idx(blk, slot):
        pltpu.make_async_copy(src_hbm.at[wid, blk], srcv.at[slot],
                              isem[slot]).wait()
        pltpu.make_async_copy(dst_hbm.at[wid, blk], dstv.at[slot],
                              isem[slot]).wait()

    def fire_gather(s, c, buf):
        pltpu.async_copy(x_hbm.at[srcv.at[s, c]], rows.at[buf], gsem[buf])

    def wait_gather(s, c, buf):
        pltpu.make_async_copy(x_hbm.at[srcv.at[s, c]], rows.at[buf],
                              gsem[buf]).wait()

    def fire_scatter(s, c, buf):
        pltpu.async_copy(rows.at[buf], agg_sh.at[dstv.at[s, c]], ssem[buf],
                         add=True)

    def wait_scatter(buf):
        pltpu.make_async_copy(rows.at[buf], agg_sh.at[sinkv.at[0]],
                              ssem[buf]).wait()

    fire_idx(0, 0)
    pltpu.sync_copy(z_hbm, agg_sh.at[pl.ds(sid * _ZROWS, _ZROWS)])
    pltpu.sync_copy(sink_hbm, sinkv)
    wait_idx(0, 0)
    for k in range(3):
        fire_gather(0, k, k)
    pltpu.async_copy(rows.at[3], agg_sh.at[sinkv.at[0]], ssem[3], add=True)
    plsc.subcore_barrier()

    @pl.loop(0, _NBLK, step=2)
    def _blocks(b0):
        for s in range(2):
            bb = b0 + s

            for c in range(_BLK):
                wait_gather(s, c, c % 4)
                fire_scatter(s, c, c % 4)
                wait_scatter((c + 3) % 4)
                if c + 3 < _BLK:
                    fire_gather(s, c + 3, (c + 3) % 4)
                if c == 1:
                    @pl.when(bb + 1 < _NBLK)
                    def _():
                        fire_idx(bb + 1, 1 - s)

            @pl.when(bb + 1 < _NBLK)
            def _():
                wait_idx(bb + 1, 1 - s)
                for k in range(3):
                    fire_gather(1 - s, k, k)

    wait_scatter(3)
    plsc.subcore_barrier()
    pltpu.sync_copy(agg_sh.at[pl.ds(sid * _ZROWS, _ZROWS)],
                    out_hbm.at[cid, pl.ds(sid * _ZROWS, _ZROWS)])


def _layer_body(h_ref, agg_ref, w1_ref, b1_ref, w2_ref, b2_ref, g_ref, be_ref, o_ref):
    u = h_ref[...] + agg_ref[0] + agg_ref[1]
    t = jnp.dot(u, w1_ref[...], preferred_element_type=jnp.float32) + b1_ref[...]
    t = jnp.where(t > 0, t, 0.01 * t)
    v = jnp.dot(t, w2_ref[...], preferred_element_type=jnp.float32) + b2_ref[...]
    v = jnp.where(v > 0, v, 0.01 * v)
    m = jnp.mean(v, axis=0, keepdims=True)
    c = v - m
    var = jnp.mean(c * c, axis=0, keepdims=True)
    o_ref[...] = c * lax.rsqrt(var + 1e-5) * g_ref[...] + be_ref[...]


_layer_call = pl.pallas_call(
    _layer_body,
    out_shape=jax.ShapeDtypeStruct((_N, _D), jnp.float32),
)


def _head_body(h_ref, batch_ref, wh1_ref, bh1_ref, wh2_ref, bh2_ref, o_ref):
    onehot = (batch_ref[...] == lax.broadcasted_iota(jnp.int32, (1, _G), 1)
              ).astype(jnp.float32)
    pooled = lax.dot_general(onehot, h_ref[...], (((0,), (0,)), ((), ())),
                             preferred_element_type=jnp.float32)
    counts = jnp.sum(onehot, axis=0)[:, None]
    pooled = pooled / jnp.maximum(counts, 1.0)
    t = jnp.dot(pooled, wh1_ref[...], preferred_element_type=jnp.float32) + bh1_ref[...]
    t = jnp.where(t > 0, t, 0.01 * t)
    o_ref[...] = jnp.dot(t, wh2_ref[...], preferred_element_type=jnp.float32) + bh2_ref[...]


_head_call = pl.pallas_call(
    _head_body,
    out_shape=jax.ShapeDtypeStruct((_G, _OUT), jnp.float32),
)


def kernel(x, edge_index, batch,
           W1_0, b1_0, W2_0, b2_0, g_0, be_0,
           W1_1, b1_1, W2_1, b2_1, g_1, be_1,
           W1_2, b1_2, W2_2, b2_2, g_2, be_2,
           Wh1, bh1, Wh2, bh2):
    src = edge_index[0]
    dst = edge_index[1]
    pad = _CH * _CHUNK - _EPT
    sink_rows = _N + (jnp.arange(_NW * pad, dtype=jnp.int32) % (_NPAD - _N))
    srcp = jnp.concatenate(
        [src.reshape(_NW, _EPT), jnp.zeros((_NW, pad), jnp.int32)], axis=1,
    ).reshape(_NW, _NBLK, _BLK, _CHUNK)
    dstp = jnp.concatenate(
        [dst.reshape(_NW, _EPT), sink_rows.reshape(_NW, pad)], axis=1,
    ).reshape(_NW, _NBLK, _BLK, _CHUNK)
    zeros = jnp.zeros((_ZROWS, _D), jnp.float32)
    sink = (_N + (jnp.arange(_CHUNK, dtype=jnp.int32) % (_NPAD - _N))
            ).reshape(1, _CHUNK)

    layers = [
        (W1_0, b1_0, W2_0, b2_0, g_0, be_0),
        (W1_1, b1_1, W2_1, b2_1, g_1, be_1),
        (W1_2, b1_2, W2_2, b2_2, g_2, be_2),
    ]
    h = x
    for (W1, b1, W2, b2, g, be) in layers:
        agg = _sc_agg(h, srcp, dstp, zeros, sink)[:, :_N, :]
        h = _layer_call(h, agg, W1, b1.reshape(1, _D), W2, b2.reshape(1, _D),
                        g.reshape(1, _D), be.reshape(1, _D))
    return _head_call(h, batch.reshape(_N, 1), Wh1, bh1.reshape(1, _D),
                      Wh2, bh2.reshape(1, _OUT))


# speedup vs baseline: 1.3506x; 1.0112x over previous
import functools

import jax
import jax.numpy as jnp
from jax import lax
from jax.experimental import pallas as pl
from jax.experimental.pallas import tpu as pltpu
from jax.experimental.pallas import tpu_sc as plsc

_N = 10000
_D = 128
_E = 320000
_G = 64
_OUT = 10

_NW = 32
_NPAD = 10112
_ZROWS = _NPAD // 16
_CHUNK = 80
_BLK = 8
_NBLK = 16
_CH = _NBLK * _BLK
_EPT = _E // _NW
_EPAD = _NW * _CH * _CHUNK

_mesh = plsc.VectorSubcoreMesh(core_axis_name="c", subcore_axis_name="s")


@functools.partial(
    pl.kernel,
    out_type=jax.ShapeDtypeStruct((2, _NPAD, _D), jnp.float32),
    mesh=_mesh,
    scratch_types=[
        pltpu.VMEM((2, _BLK, _CHUNK), jnp.int32),
        pltpu.VMEM((2, _BLK, _CHUNK), jnp.int32),
        pltpu.VMEM((4, _CHUNK, _D), jnp.float32),
        pltpu.VMEM((1, _CHUNK), jnp.int32),
        pltpu.VMEM_SHARED((_NPAD, _D), jnp.float32),
        (pltpu.SemaphoreType.DMA, pltpu.SemaphoreType.DMA,
         pltpu.SemaphoreType.DMA, pltpu.SemaphoreType.DMA),
        (pltpu.SemaphoreType.DMA, pltpu.SemaphoreType.DMA,
         pltpu.SemaphoreType.DMA, pltpu.SemaphoreType.DMA),
        (pltpu.SemaphoreType.DMA, pltpu.SemaphoreType.DMA),
    ],
)
def _sc_agg(x_hbm, src_hbm, dst_hbm, z_hbm, sink_hbm, out_hbm,
            srcv, dstv, rows, sinkv, agg_sh, gsem, ssem, isem):
    cid = lax.axis_index("c")
    sid = lax.axis_index("s")
    wid = sid * 2 + cid

    def fire_idx(blk, slot):
        pltpu.async_copy(src_hbm.at[wid, blk], srcv.at[slot], isem[slot])
        pltpu.async_copy(dst_hbm.at[wid, blk], dstv.at[slot], isem[slot])

    def wait_idx(blk, slot):
        pltpu.make_async_copy(src_hbm.at[wid, blk], srcv.at[slot],
                              isem[slot]).wait()
        pltpu.make_async_copy(dst_hbm.at[wid, blk], dstv.at[slot],
                              isem[slot]).wait()

    def fire_gather(s, c, buf):
        pltpu.async_copy(x_hbm.at[srcv.at[s, c]], rows.at[buf], gsem[buf])

    def wait_gather(s, c, buf):
        pltpu.make_async_copy(x_hbm.at[srcv.at[s, c]], rows.at[buf],
                              gsem[buf]).wait()

    def fire_scatter(s, c, buf):
        pltpu.async_copy(rows.at[buf], agg_sh.at[dstv.at[s, c]], ssem[buf],
                         add=True)

    def wait_scatter(buf):
        pltpu.make_async_copy(rows.at[buf], agg_sh.at[sinkv.at[0]],
                              ssem[buf]).wait()

    fire_idx(0, 0)
    pltpu.sync_copy(z_hbm, agg_sh.at[pl.ds(sid * _ZROWS, _ZROWS)])
    pltpu.sync_copy(sink_hbm, sinkv)
    wait_idx(0, 0)
    for k in range(3):
        fire_gather(0, k, k)
    pltpu.async_copy(rows.at[3], agg_sh.at[sinkv.at[0]], ssem[3], add=True)
    plsc.subcore_barrier()

    @pl.loop(0, _NBLK, step=2)
    def _blocks(b0):
        for s in range(2):
            bb = b0 + s

            for c in range(_BLK):
                wait_gather(s, c, c % 4)
                fire_scatter(s, c, c % 4)
                wait_scatter((c + 3) % 4)
                if c + 3 < _BLK:
                    fire_gather(s, c + 3, (c + 3) % 4)
                if c == 1:
                    @pl.when(bb + 1 < _NBLK)
                    def _():
                        fire_idx(bb + 1, 1 - s)

            @pl.when(bb + 1 < _NBLK)
            def _():
                wait_idx(bb + 1, 1 - s)
                for k in range(3):
                    fire_gather(1 - s, k, k)

    wait_scatter(3)
    plsc.subcore_barrier()
    pltpu.sync_copy(agg_sh.at[pl.ds(sid * _ZROWS, _ZROWS)],
                    out_hbm.at[cid, pl.ds(sid * _ZROWS, _ZROWS)])


def _layer_body(h_ref, agg_ref, w1_ref, b1_ref, w2_ref, b2_ref, g_ref, be_ref, o_ref):
    u = h_ref[...] + agg_ref[0] + agg_ref[1]
    t = jnp.dot(u, w1_ref[...], preferred_element_type=jnp.float32) + b1_ref[...]
    t = jnp.where(t > 0, t, 0.01 * t)
    v = jnp.dot(t, w2_ref[...], preferred_element_type=jnp.float32) + b2_ref[...]
    v = jnp.where(v > 0, v, 0.01 * v)
    m = jnp.mean(v, axis=0, keepdims=True)
    c = v - m
    var = jnp.mean(c * c, axis=0, keepdims=True)
    o_ref[...] = c * lax.rsqrt(var + 1e-5) * g_ref[...] + be_ref[...]


_layer_call = pl.pallas_call(
    _layer_body,
    out_shape=jax.ShapeDtypeStruct((_N, _D), jnp.float32),
)


def _head_body(h_ref, batch_ref, wh1_ref, bh1_ref, wh2_ref, bh2_ref, o_ref):
    onehot = (batch_ref[...] == lax.broadcasted_iota(jnp.int32, (1, _G), 1)
              ).astype(jnp.float32)
    pooled = lax.dot_general(onehot, h_ref[...], (((0,), (0,)), ((), ())),
                             preferred_element_type=jnp.float32)
    counts = jnp.sum(onehot, axis=0)[:, None]
    pooled = pooled / jnp.maximum(counts, 1.0)
    t = jnp.dot(pooled, wh1_ref[...], preferred_element_type=jnp.float32) + bh1_ref[...]
    t = jnp.where(t > 0, t, 0.01 * t)
    o_ref[...] = jnp.dot(t, wh2_ref[...], preferred_element_type=jnp.float32) + bh2_ref[...]


_head_call = pl.pallas_call(
    _head_body,
    out_shape=jax.ShapeDtypeStruct((_G, _OUT), jnp.float32),
)


def kernel(x, edge_index, batch,
           W1_0, b1_0, W2_0, b2_0, g_0, be_0,
           W1_1, b1_1, W2_1, b2_1, g_1, be_1,
           W1_2, b1_2, W2_2, b2_2, g_2, be_2,
           Wh1, bh1, Wh2, bh2):
    src = edge_index[0]
    dst = edge_index[1]
    pad = _CH * _CHUNK - _EPT
    sink_rows = _N + (jnp.arange(_NW * pad, dtype=jnp.int32) % (_NPAD - _N))
    srcp = jnp.concatenate(
        [src.reshape(_NW, _EPT), jnp.zeros((_NW, pad), jnp.int32)], axis=1,
    ).reshape(_NW, _NBLK, _BLK, _CHUNK)
    dstp = jnp.concatenate(
        [dst.reshape(_NW, _EPT), sink_rows.reshape(_NW, pad)], axis=1,
    ).reshape(_NW, _NBLK, _BLK, _CHUNK)
    zeros = jnp.zeros((_ZROWS, _D), jnp.float32)
    sink = (_N + (jnp.arange(_CHUNK, dtype=jnp.int32) % (_NPAD - _N))
            ).reshape(1, _CHUNK)

    layers = [
        (W1_0, b1_0, W2_0, b2_0, g_0, be_0),
        (W1_1, b1_1, W2_1, b2_1, g_1, be_1),
        (W1_2, b1_2, W2_2, b2_2, g_2, be_2),
    ]
    h = x
    for (W1, b1, W2, b2, g, be) in layers:
        agg = _sc_agg(h, srcp, dstp, zeros, sink)[:, :_N, :]
        h = _layer_call(h, agg, W1, b1.reshape(1, _D), W2, b2.reshape(1, _D),
                        g.reshape(1, _D), be.reshape(1, _D))
    return _head_call(h, batch.reshape(_N, 1), Wh1, bh1.reshape(1, _D),
                      Wh2, bh2.reshape(1, _OUT))


# speedup vs baseline: 1.3682x; 1.0130x over previous
import functools

import jax
import jax.numpy as jnp
from jax import lax
from jax.experimental import pallas as pl
from jax.experimental.pallas import tpu as pltpu
from jax.experimental.pallas import tpu_sc as plsc

_N = 10000
_D = 128
_E = 320000
_G = 64
_OUT = 10

_NW = 32
_NPAD = 10112
_ZROWS = _NPAD // 16
_CHUNK = 80
_BLK = 16
_NBLK = 8
_CH = _NBLK * _BLK
_EPT = _E // _NW
_EPAD = _NW * _CH * _CHUNK

_mesh = plsc.VectorSubcoreMesh(core_axis_name="c", subcore_axis_name="s")


@functools.partial(
    pl.kernel,
    out_type=jax.ShapeDtypeStruct((2, _NPAD, _D), jnp.float32),
    mesh=_mesh,
    scratch_types=[
        pltpu.VMEM((2, _BLK, _CHUNK), jnp.int32),
        pltpu.VMEM((2, _BLK, _CHUNK), jnp.int32),
        pltpu.VMEM((4, _CHUNK, _D), jnp.float32),
        pltpu.VMEM((1, _CHUNK), jnp.int32),
        pltpu.VMEM_SHARED((_NPAD, _D), jnp.float32),
        (pltpu.SemaphoreType.DMA, pltpu.SemaphoreType.DMA,
         pltpu.SemaphoreType.DMA, pltpu.SemaphoreType.DMA),
        (pltpu.SemaphoreType.DMA, pltpu.SemaphoreType.DMA,
         pltpu.SemaphoreType.DMA, pltpu.SemaphoreType.DMA),
        (pltpu.SemaphoreType.DMA, pltpu.SemaphoreType.DMA),
    ],
)
def _sc_agg(x_hbm, src_hbm, dst_hbm, z_hbm, sink_hbm, out_hbm,
            srcv, dstv, rows, sinkv, agg_sh, gsem, ssem, isem):
    cid = lax.axis_index("c")
    sid = lax.axis_index("s")
    wid = sid * 2 + cid

    def fire_idx(blk, slot):
        pltpu.async_copy(src_hbm.at[wid, blk], srcv.at[slot], isem[slot])
        pltpu.async_copy(dst_hbm.at[wid, blk], dstv.at[slot], isem[slot])

    def wait_idx(blk, slot):
        pltpu.make_async_copy(src_hbm.at[wid, blk], srcv.at[slot],
                              isem[slot]).wait()
        pltpu.make_async_copy(dst_hbm.at[wid, blk], dstv.at[slot],
                              isem[slot]).wait()

    def fire_gather(s, c, buf):
        pltpu.async_copy(x_hbm.at[srcv.at[s, c]], rows.at[buf], gsem[buf])

    def wait_gather(s, c, buf):
        pltpu.make_async_copy(x_hbm.at[srcv.at[s, c]], rows.at[buf],
                              gsem[buf]).wait()

    def fire_scatter(s, c, buf):
        pltpu.async_copy(rows.at[buf], agg_sh.at[dstv.at[s, c]], ssem[buf],
                         add=True)

    def wait_scatter(buf):
        pltpu.make_async_copy(rows.at[buf], agg_sh.at[sinkv.at[0]],
                              ssem[buf]).wait()

    fire_idx(0, 0)
    pltpu.sync_copy(z_hbm, agg_sh.at[pl.ds(sid * _ZROWS, _ZROWS)])
    pltpu.sync_copy(sink_hbm, sinkv)
    wait_idx(0, 0)
    for k in range(3):
        fire_gather(0, k, k)
    pltpu.async_copy(rows.at[3], agg_sh.at[sinkv.at[0]], ssem[3], add=True)
    plsc.subcore_barrier()

    @pl.loop(0, _NBLK, step=2)
    def _blocks(b0):
        for s in range(2):
            bb = b0 + s

            for c in range(_BLK):
                wait_gather(s, c, c % 4)
                fire_scatter(s, c, c % 4)
                wait_scatter((c + 3) % 4)
                if c + 3 < _BLK:
                    fire_gather(s, c + 3, (c + 3) % 4)
                if c == 1:
                    @pl.when(bb + 1 < _NBLK)
                    def _():
                        fire_idx(bb + 1, 1 - s)

            @pl.when(bb + 1 < _NBLK)
            def _():
                wait_idx(bb + 1, 1 - s)
                for k in range(3):
                    fire_gather(1 - s, k, k)

    wait_scatter(3)
    plsc.subcore_barrier()
    pltpu.sync_copy(agg_sh.at[pl.ds(sid * _ZROWS, _ZROWS)],
                    out_hbm.at[cid, pl.ds(sid * _ZROWS, _ZROWS)])


def _layer_body(h_ref, agg_ref, w1_ref, b1_ref, w2_ref, b2_ref, g_ref, be_ref, o_ref):
    u = h_ref[...] + agg_ref[0] + agg_ref[1]
    t = jnp.dot(u, w1_ref[...], preferred_element_type=jnp.float32) + b1_ref[...]
    t = jnp.where(t > 0, t, 0.01 * t)
    v = jnp.dot(t, w2_ref[...], preferred_element_type=jnp.float32) + b2_ref[...]
    v = jnp.where(v > 0, v, 0.01 * v)
    m = jnp.mean(v, axis=0, keepdims=True)
    c = v - m
    var = jnp.mean(c * c, axis=0, keepdims=True)
    o_ref[...] = c * lax.rsqrt(var + 1e-5) * g_ref[...] + be_ref[...]


_layer_call = pl.pallas_call(
    _layer_body,
    out_shape=jax.ShapeDtypeStruct((_N, _D), jnp.float32),
)


def _head_body(h_ref, batch_ref, wh1_ref, bh1_ref, wh2_ref, bh2_ref, o_ref):
    onehot = (batch_ref[...] == lax.broadcasted_iota(jnp.int32, (1, _G), 1)
              ).astype(jnp.float32)
    pooled = lax.dot_general(onehot, h_ref[...], (((0,), (0,)), ((), ())),
                             preferred_element_type=jnp.float32)
    counts = jnp.sum(onehot, axis=0)[:, None]
    pooled = pooled / jnp.maximum(counts, 1.0)
    t = jnp.dot(pooled, wh1_ref[...], preferred_element_type=jnp.float32) + bh1_ref[...]
    t = jnp.where(t > 0, t, 0.01 * t)
    o_ref[...] = jnp.dot(t, wh2_ref[...], preferred_element_type=jnp.float32) + bh2_ref[...]


_head_call = pl.pallas_call(
    _head_body,
    out_shape=jax.ShapeDtypeStruct((_G, _OUT), jnp.float32),
)


def kernel(x, edge_index, batch,
           W1_0, b1_0, W2_0, b2_0, g_0, be_0,
           W1_1, b1_1, W2_1, b2_1, g_1, be_1,
           W1_2, b1_2, W2_2, b2_2, g_2, be_2,
           Wh1, bh1, Wh2, bh2):
    src = edge_index[0]
    dst = edge_index[1]
    pad = _CH * _CHUNK - _EPT
    sink_rows = _N + (jnp.arange(_NW * pad, dtype=jnp.int32) % (_NPAD - _N))
    srcp = jnp.concatenate(
        [src.reshape(_NW, _EPT), jnp.zeros((_NW, pad), jnp.int32)], axis=1,
    ).reshape(_NW, _NBLK, _BLK, _CHUNK)
    dstp = jnp.concatenate(
        [dst.reshape(_NW, _EPT), sink_rows.reshape(_NW, pad)], axis=1,
    ).reshape(_NW, _NBLK, _BLK, _CHUNK)
    zeros = jnp.zeros((_ZROWS, _D), jnp.float32)
    sink = (_N + (jnp.arange(_CHUNK, dtype=jnp.int32) % (_NPAD - _N))
            ).reshape(1, _CHUNK)

    layers = [
        (W1_0, b1_0, W2_0, b2_0, g_0, be_0),
        (W1_1, b1_1, W2_1, b2_1, g_1, be_1),
        (W1_2, b1_2, W2_2, b2_2, g_2, be_2),
    ]
    h = x
    for (W1, b1, W2, b2, g, be) in layers:
        agg = _sc_agg(h, srcp, dstp, zeros, sink)[:, :_N, :]
        h = _layer_call(h, agg, W1, b1.reshape(1, _D), W2, b2.reshape(1, _D),
                        g.reshape(1, _D), be.reshape(1, _D))
    return _head_call(h, batch.reshape(_N, 1), Wh1, bh1.reshape(1, _D),
                      Wh2, bh2.reshape(1, _OUT))
